# baseline (speedup 1.0000x reference)
import functools

import jax
import jax.numpy as jnp
from jax import lax
from jax.experimental import pallas as pl
from jax.experimental.pallas import tpu as pltpu
from jax.experimental.pallas import tpu_sc as plsc

_NC = 2
_NS = 16
_NW = _NC * _NS
_L = 16

_BATCH = 16384
_D = 64
_BPW = _BATCH // _NW
_CHUNK = 256
_NCHUNK = _BPW // _CHUNK
_GROUPS = _CHUNK // _L


def _sc_body(u_hbm, i_hbm, mu_hbm, bu_hbm, bi_hbm, p_hbm, q_hbm, out_hbm,
             uidx_v, iidx_v, idxp_v, idxq_v, pu_v, qi_v, bu_v, bi_v, mu_v,
             out_v, sem):
    wid = lax.axis_index("s") * _NC + lax.axis_index("c")
    base = wid * _BPW
    pltpu.sync_copy(mu_hbm, mu_v)
    mu_vec = mu_v[...]

    for chunk in range(_NCHUNK):
        cbase = base + chunk * _CHUNK
        pltpu.sync_copy(u_hbm.at[pl.ds(cbase, _CHUNK)], uidx_v)
        pltpu.sync_copy(i_hbm.at[pl.ds(cbase, _CHUNK)], iidx_v)
        bias_cps = [
            pltpu.async_copy(bu_hbm.at[uidx_v], bu_v, sem),
            pltpu.async_copy(bi_hbm.at[iidx_v], bi_v, sem),
        ]

        def build(g, carry):
            gb = pl.multiple_of(g * _L, _L)
            sl = pl.ds(gb, _L)
            up = uidx_v[sl] * _D
            iq = iidx_v[sl] * _D
            for j in range(_D):
                idxp_v[pl.ds(j * _CHUNK + gb, _L)] = up + j
                idxq_v[pl.ds(j * _CHUNK + gb, _L)] = iq + j
            return carry

        lax.fori_loop(0, _GROUPS, build, 0)

        cps = [
            pltpu.async_copy(p_hbm.at[idxp_v], pu_v, sem),
            pltpu.async_copy(q_hbm.at[idxq_v], qi_v, sem),
        ]
        for cp in cps + bias_cps:
            cp.wait()

        def group(g, carry):
            gb = pl.multiple_of(g * _L, _L)
            sl = pl.ds(gb, _L)
            acc = jnp.zeros((_L,), jnp.float32)
            for j in range(_D):
                fsl = pl.ds(j * _CHUNK + gb, _L)
                acc = acc + pu_v[fsl] * qi_v[fsl]
            out_v[sl] = mu_vec + bu_v[sl] + bi_v[sl] + acc
            return carry

        lax.fori_loop(0, _GROUPS, group, 0)
        pltpu.sync_copy(out_v, out_hbm.at[pl.ds(cbase, _CHUNK)])


def kernel(u_idx, i_idx, mu, b_u, b_i, P, Q):
    u_idx = u_idx.astype(jnp.int32)
    i_idx = i_idx.astype(jnp.int32)
    mu_vec = jnp.broadcast_to(mu.astype(jnp.float32), (_L,))
    mesh = plsc.VectorSubcoreMesh(core_axis_name="c", subcore_axis_name="s")
    run = functools.partial(
        pl.kernel,
        mesh=mesh,
        compiler_params=pltpu.CompilerParams(needs_layout_passes=False),
        out_type=jax.ShapeDtypeStruct((_BATCH,), jnp.float32),
        scratch_types=[
            pltpu.VMEM((_CHUNK,), jnp.int32),
            pltpu.VMEM((_CHUNK,), jnp.int32),
            pltpu.VMEM((_CHUNK * _D,), jnp.int32),
            pltpu.VMEM((_CHUNK * _D,), jnp.int32),
            pltpu.VMEM((_CHUNK * _D,), jnp.float32),
            pltpu.VMEM((_CHUNK * _D,), jnp.float32),
            pltpu.VMEM((_CHUNK,), jnp.float32),
            pltpu.VMEM((_CHUNK,), jnp.float32),
            pltpu.VMEM((_L,), jnp.float32),
            pltpu.VMEM((_CHUNK,), jnp.float32),
            pltpu.SemaphoreType.DMA,
        ],
    )(_sc_body)
    return run(u_idx, i_idx, mu_vec, b_u, b_i,
               P.reshape(-1), Q.reshape(-1))


# speedup vs baseline: 1.1484x; 1.1484x over previous
import functools

import jax
import jax.numpy as jnp
from jax import lax
from jax.experimental import pallas as pl
from jax.experimental.pallas import tpu as pltpu
from jax.experimental.pallas import tpu_sc as plsc

_NC = 2
_NS = 16
_NW = _NC * _NS
_L = 16

_BATCH = 16384
_D = 64
_W = 128
_BPW = _BATCH // _NW
_CHUNK = 256
_NCHUNK = _BPW // _CHUNK
_GROUPS = _CHUNK // _L


def _sc_body(u_hbm, i_hbm, mu_hbm, bu_hbm, bi_hbm, p_hbm, q_hbm, out_hbm,
             uidx_v, iidx_v, pu_v, qi_v, bu_v, bi_v, mu_v, out_v, sem):
    wid = lax.axis_index("s") * _NC + lax.axis_index("c")
    base = wid * _BPW
    pltpu.sync_copy(mu_hbm, mu_v)
    mu_vec = mu_v[...]

    lane_iota = lax.iota(jnp.int32, _L)
    lane_masks = [lane_iota == r for r in range(_L)]

    for chunk in range(_NCHUNK):
        cbase = base + chunk * _CHUNK
        csl = pl.ds(cbase, _CHUNK)
        pltpu.sync_copy(u_hbm.at[csl], uidx_v)
        pltpu.sync_copy(i_hbm.at[csl], iidx_v)
        cps = [
            pltpu.async_copy(bu_hbm.at[uidx_v], bu_v, sem),
            pltpu.async_copy(bi_hbm.at[iidx_v], bi_v, sem),
            pltpu.async_copy(p_hbm.at[uidx_v], pu_v, sem),
            pltpu.async_copy(q_hbm.at[iidx_v], qi_v, sem),
        ]
        for cp in cps:
            cp.wait()

        def group(g, carry):
            gb = pl.multiple_of(g * _L, _L)
            sl = pl.ds(gb, _L)
            dots = jnp.zeros((_L,), jnp.float32)
            for r in range(_L):
                b = gb + r
                acc = pu_v[b, pl.ds(0, _L)] * qi_v[b, pl.ds(0, _L)]
                for c in range(1, _D // _L):
                    acc = acc + (pu_v[b, pl.ds(c * _L, _L)] *
                                 qi_v[b, pl.ds(c * _L, _L)])
                dots = jnp.where(lane_masks[r], jnp.sum(acc), dots)
            out_v[sl] = mu_vec + bu_v[sl] + bi_v[sl] + dots
            return carry

        lax.fori_loop(0, _GROUPS, group, 0)
        pltpu.sync_copy(out_v, out_hbm.at[csl])


def kernel(u_idx, i_idx, mu, b_u, b_i, P, Q):
    u_idx = u_idx.astype(jnp.int32)
    i_idx = i_idx.astype(jnp.int32)
    mu_vec = jnp.broadcast_to(mu.astype(jnp.float32), (_L,))
    P_w = jnp.pad(P, ((0, 0), (0, _W - _D)))
    Q_w = jnp.pad(Q, ((0, 0), (0, _W - _D)))
    mesh = plsc.VectorSubcoreMesh(core_axis_name="c", subcore_axis_name="s")
    run = functools.partial(
        pl.kernel,
        mesh=mesh,
        compiler_params=pltpu.CompilerParams(
            needs_layout_passes=False, use_tc_tiling_on_sc=True),
        out_type=jax.ShapeDtypeStruct((_BATCH,), jnp.float32),
        scratch_types=[
            pltpu.VMEM((_CHUNK,), jnp.int32),
            pltpu.VMEM((_CHUNK,), jnp.int32),
            pltpu.VMEM((_CHUNK, _W), jnp.float32),
            pltpu.VMEM((_CHUNK, _W), jnp.float32),
            pltpu.VMEM((_CHUNK,), jnp.float32),
            pltpu.VMEM((_CHUNK,), jnp.float32),
            pltpu.VMEM((_L,), jnp.float32),
            pltpu.VMEM((_CHUNK,), jnp.float32),
            pltpu.SemaphoreType.DMA,
        ],
    )(_sc_body)
    return run(u_idx, i_idx, mu_vec, b_u, b_i, P_w, Q_w)
